# baseline (speedup 1.0000x reference)
import jax
import jax.numpy as jnp
from jax import lax
from jax.experimental import pallas as pl
from jax.experimental.pallas import tpu as pltpu
import jax.experimental.pallas.tpu_sc as plsc

N = 10000
E = 320000
D = 128
NHEAD = 8
HC = 16
HD = D // 2
HH = NHEAD // 2

NC = 2
NS = 16
LB = 16

EB = 128
E_TOT = E + N
NBUF = 3
NSC = 2
CH = 6
KBLK = 162
NCHUNK = KBLK // CH
E_PAD = KBLK * NS * EB
N_ACC = N + 8
AW = LB + HD

ROWS_PER_TILE = N // NS
FLUSH_ROWS = (N // NS) // 8 * 8

NB = 1000
GRID = N // NB


def _att_mats(a):
    h, hc = a.shape
    mats = []
    for c in range(NC):
        A = jnp.zeros((D, LB), jnp.float32)
        if h == 1:
            A = A.at[:, :HH].set(jnp.tile(a.reshape(D, 1), (1, HH)))
        else:
            rows = jnp.arange(D)
            cols = jnp.repeat(jnp.arange(h), hc) - c * HH
            keep = (cols >= 0) & (cols < HH)
            A = A.at[rows, jnp.where(keep, cols, HH)].set(
                jnp.where(keep, a.reshape(-1), 0.0))
        mats.append(A)
    return mats


def _sel_mats(heads):
    ps, rs = [], []
    for c in range(NC):
        P = jnp.zeros((AW, D), jnp.float32)
        P = P.at[LB:, c * HD:(c + 1) * HD].set(jnp.eye(HD, dtype=jnp.float32))
        ps.append(P)
        R = jnp.zeros((AW, D), jnp.float32)
        if heads == 1:
            R = R.at[0, c * HD:(c + 1) * HD].set(1.0)
        else:
            for l in range(HH):
                R = R.at[l, c * HD + l * HC:c * HD + (l + 1) * HC].set(1.0)
        rs.append(R)
    return ps, rs


def _split_store(xl, sa0, sa1, da0, da1, xls_ref, sas_ref, das_ref):
    xls_ref[...] = jnp.stack([xl[:, :HD], xl[:, HD:]])
    sas_ref[...] = jnp.stack([sa0, sa1])
    das_ref[...] = jnp.stack([da0, da1])


def _tc_first_body(x_ref, w_ref, wr_ref, as0_ref, as1_ref, ad0_ref, ad1_ref,
                   xls_ref, sas_ref, das_ref, res_ref):
    xv = x_ref[...]
    xl = jnp.dot(xv, w_ref[...].T, preferred_element_type=jnp.float32)
    _split_store(
        xl,
        jnp.dot(xl, as0_ref[...], preferred_element_type=jnp.float32),
        jnp.dot(xl, as1_ref[...], preferred_element_type=jnp.float32),
        jnp.dot(xl, ad0_ref[...], preferred_element_type=jnp.float32),
        jnp.dot(xl, ad1_ref[...], preferred_element_type=jnp.float32),
        xls_ref, sas_ref, das_ref)
    res_ref[...] = jnp.dot(xv, wr_ref[...].T, preferred_element_type=jnp.float32)


def _tc_comb_body(np_ref, res_ref, bvec_ref, scale_ref, shift_ref,
                  p0_ref, p1_ref, r0_ref, r1_ref,
                  w_ref, wr_ref, as0_ref, as1_ref, ad0_ref, ad1_ref,
                  xls_ref, sas_ref, das_ref, res2_ref):
    a0 = np_ref[0]
    a1 = np_ref[1]
    num = (jnp.dot(a0, p0_ref[...], preferred_element_type=jnp.float32)
           + jnp.dot(a1, p1_ref[...], preferred_element_type=jnp.float32))
    denb = (jnp.dot(a0, r0_ref[...], preferred_element_type=jnp.float32)
            + jnp.dot(a1, r1_ref[...], preferred_element_type=jnp.float32)
            + 1e-16)
    hv = num / denb + res_ref[...] + bvec_ref[...]
    hv = jnp.maximum(hv * scale_ref[...] + shift_ref[...], 0.0)
    xl = jnp.dot(hv, w_ref[...].T, preferred_element_type=jnp.float32)
    _split_store(
        xl,
        jnp.dot(xl, as0_ref[...], preferred_element_type=jnp.float32),
        jnp.dot(xl, as1_ref[...], preferred_element_type=jnp.float32),
        jnp.dot(xl, ad0_ref[...], preferred_element_type=jnp.float32),
        jnp.dot(xl, ad1_ref[...], preferred_element_type=jnp.float32),
        xls_ref, sas_ref, das_ref)
    res2_ref[...] = jnp.dot(hv, wr_ref[...].T, preferred_element_type=jnp.float32)


def _tc_final_body(np_ref, res_ref, bvec_ref, p0_ref, p1_ref, r0_ref, r1_ref,
                   out_ref):
    a0 = np_ref[0]
    a1 = np_ref[1]
    num = (jnp.dot(a0, p0_ref[...], preferred_element_type=jnp.float32)
           + jnp.dot(a1, p1_ref[...], preferred_element_type=jnp.float32))
    denb = (jnp.dot(a0, r0_ref[...], preferred_element_type=jnp.float32)
            + jnp.dot(a1, r1_ref[...], preferred_element_type=jnp.float32)
            + 1e-16)
    out_ref[...] = num / denb + res_ref[...] + bvec_ref[...]


_FULL = lambda shape: pl.BlockSpec(shape, lambda i: tuple(0 for _ in shape))
_ROWS = pl.BlockSpec((NB, D), lambda i: (i, 0))
_PARTS = pl.BlockSpec((NC, NB, AW), lambda i: (0, i, 0))
_SPLIT = pl.BlockSpec((NC, NB, HD), lambda i: (0, i, 0))
_SPLIT16 = pl.BlockSpec((NC, NB, LB), lambda i: (0, i, 0))

_SPLIT_OUT_SHAPES = [jax.ShapeDtypeStruct((NC, N, HD), jnp.float32),
                     jax.ShapeDtypeStruct((NC, N, LB), jnp.float32),
                     jax.ShapeDtypeStruct((NC, N, LB), jnp.float32),
                     jax.ShapeDtypeStruct((N, D), jnp.float32)]


def _tc_first(x, w, wr, a_s, a_d):
    return pl.pallas_call(
        _tc_first_body,
        grid=(GRID,),
        in_specs=[_ROWS, _FULL((D, D)), _FULL((D, D)),
                  _FULL((D, LB)), _FULL((D, LB)),
                  _FULL((D, LB)), _FULL((D, LB))],
        out_specs=[_SPLIT, _SPLIT16, _SPLIT16, _ROWS],
        out_shape=_SPLIT_OUT_SHAPES,
    )(x, w, wr, a_s[0], a_s[1], a_d[0], a_d[1])


def _tc_comb(npart, res, bvec, scale, shift, p, r, w, wr, a_s, a_d):
    return pl.pallas_call(
        _tc_comb_body,
        grid=(GRID,),
        in_specs=[_PARTS, _ROWS, _FULL((1, D)), _FULL((1, D)), _FULL((1, D)),
                  _FULL((AW, D)), _FULL((AW, D)), _FULL((AW, D)),
                  _FULL((AW, D)),
                  _FULL((D, D)), _FULL((D, D)),
                  _FULL((D, LB)), _FULL((D, LB)),
                  _FULL((D, LB)), _FULL((D, LB))],
        out_specs=[_SPLIT, _SPLIT16, _SPLIT16, _ROWS],
        out_shape=_SPLIT_OUT_SHAPES,
    )(npart, res, bvec, scale, shift, p[0], p[1], r[0], r[1],
      w, wr, a_s[0], a_s[1], a_d[0], a_d[1])


def _tc_final(npart, res, bvec, p, r):
    return pl.pallas_call(
        _tc_final_body,
        grid=(GRID,),
        in_specs=[_PARTS, _ROWS, _FULL((1, D)),
                  _FULL((AW, D)), _FULL((AW, D)),
                  _FULL((AW, D)), _FULL((AW, D))],
        out_specs=_ROWS,
        out_shape=jax.ShapeDtypeStruct((N, D), jnp.float32),
    )(npart, res, bvec, p[0], p[1], r[0], r[1])


def _sc_edge_body(eidx_hbm, sa_hbm, da_hbm, xl_hbm,
                  np_out,
                  idx_v, sa_v, da_v, xg_v, wx_v,
                  semg0, semg1, semg2, sems0, sems1, semi,
                  acc):
    cid = lax.axis_index("c")
    sid = lax.axis_index("s")
    semg = (semg0, semg1, semg2)
    sems = (sems0, sems1)
    cofs = cid * N

    z16 = lax.iota(jnp.int32, LB).astype(jnp.float32) * 0.0

    for b in range(NSC):
        def _zx(i, _, b=b):
            wx_v[b, i // (AW // LB), pl.ds((i % (AW // LB)) * LB, LB)] = z16
            return 0
        lax.fori_loop(0, EB * (AW // LB), _zx, 0)

    base_row = sid * ROWS_PER_TILE
    for r in range(4):
        pltpu.sync_copy(wx_v.at[0], acc.at[pl.ds(base_row + EB * r, EB)])
    rem = ROWS_PER_TILE - 4 * EB
    pltpu.sync_copy(wx_v.at[0, pl.ds(0, rem)],
                    acc.at[pl.ds(base_row + 4 * EB, rem)])
    plsc.subcore_barrier()

    def _fetch_idx(c, p):
        pltpu.async_copy(eidx_hbm.at[sid, pl.ds(c * CH, CH)], idx_v.at[p],
                         semi)

    def _drain_idx(p):
        pltpu.make_async_copy(eidx_hbm.at[sid, pl.ds(0, CH)], idx_v.at[p],
                              semi).wait()

    def _fixup_idx(p):
        for jj in range(CH):
            for rr in range(2):
                def _fx(q, _, jj=jj, rr=rr):
                    sl = pl.ds(q * LB, LB)
                    idx_v[p, jj, rr, sl] = idx_v[p, jj, rr, sl] + cofs
                    return 0
                lax.fori_loop(0, EB // LB, _fx, 0)

    def _issue_scat(b, p, j):
        pltpu.async_copy(wx_v.at[b], acc.at[idx_v.at[p, j, 2]],
                         sems[b], add=True)

    def _drain_scat(b):
        pltpu.make_async_copy(wx_v.at[b], acc.at[idx_v.at[0, 0, 2]],
                              sems[b]).wait()

    def _issue_gath(b, p, j):
        pltpu.async_copy(sa_hbm.at[idx_v.at[p, j, 0]], sa_v.at[b], semg[b])
        pltpu.async_copy(da_hbm.at[idx_v.at[p, j, 1]], da_v.at[b], semg[b])
        pltpu.async_copy(xl_hbm.at[idx_v.at[p, j, 0]], xg_v.at[b], semg[b])

    def _drain_gath(b):
        pltpu.make_async_copy(sa_hbm.at[idx_v.at[0, 0, 0]], sa_v.at[b],
                              semg[b]).wait()
        pltpu.make_async_copy(da_hbm.at[idx_v.at[0, 0, 1]], da_v.at[b],
                              semg[b]).wait()
        pltpu.make_async_copy(xl_hbm.at[idx_v.at[0, 0, 0]], xg_v.at[b],
                              semg[b]).wait()

    def _compute(bg, bs):
        def _edge(e, _):
            v = sa_v[bg, e, :] + da_v[bg, e, :]
            v = jnp.maximum(v, 0.2 * v)
            v = jnp.exp(v)
            wx_v[bs, e, pl.ds(0, LB)] = v
            for h in range(HH):
                s = v[h]
                wx_v[bs, e, pl.ds(LB + h * HC, HC)] = (
                    xg_v[bg, e, pl.ds(h * HC, HC)] * s)
            return 0
        lax.fori_loop(0, EB, _edge, 0, unroll=4)

    _fetch_idx(0, 0)
    _drain_idx(0)
    _fixup_idx(0)
    for b in range(NSC):
        _issue_scat(b, 0, 0)
    _issue_gath(0, 0, 0)
    _issue_gath(1, 0, 1)

    def _chunk(c, _):
        p = lax.rem(c, 2)
        pn = lax.rem(c + 1, 2)
        cn = jnp.minimum(c + 1, NCHUNK - 1)
        for j in range(CH):
            k_bg = j % NBUF
            k_bs = j % NSC
            _drain_gath(k_bg)
            _drain_scat(k_bs)
            _compute(k_bg, k_bs)
            _issue_scat(k_bs, p, j)
            if j == 1:
                _fetch_idx(cn, pn)
            if j == 3:
                _drain_idx(pn)
                _fixup_idx(pn)
            if j < CH - 2:
                _issue_gath((j + 2) % NBUF, p, j + 2)
            else:
                _issue_gath((j + 2) % NBUF, pn, j + 2 - CH)
        return 0
    lax.fori_loop(0, NCHUNK, _chunk, 0)

    _drain_scat(0)
    _drain_scat(1)
    _drain_gath(0)
    _drain_gath(1)
    plsc.subcore_barrier()

    rs = pl.ds(sid * FLUSH_ROWS, FLUSH_ROWS)
    pltpu.sync_copy(acc.at[rs], np_out.at[cid, rs])

    @pl.when(sid == NS - 1)
    def _tail():
        rs2 = pl.ds(NS * FLUSH_ROWS, N - NS * FLUSH_ROWS)
        pltpu.sync_copy(acc.at[rs2], np_out.at[cid, rs2])


def _sc_edge():
    return pl.kernel(
        _sc_edge_body,
        out_type=jax.ShapeDtypeStruct((NC, N, AW), jnp.float32),
        mesh=plsc.VectorSubcoreMesh(core_axis_name="c", subcore_axis_name="s",
                                    num_cores=NC, num_subcores=NS),
        compiler_params=pltpu.CompilerParams(use_tc_tiling_on_sc=False),
        scratch_types=[
            pltpu.VMEM((2, CH, 3, EB), jnp.int32),
            pltpu.VMEM((NBUF, EB, LB), jnp.float32),
            pltpu.VMEM((NBUF, EB, LB), jnp.float32),
            pltpu.VMEM((NBUF, EB, HD), jnp.float32),
            pltpu.VMEM((NSC, EB, AW), jnp.float32),
            pltpu.SemaphoreType.DMA,
            pltpu.SemaphoreType.DMA,
            pltpu.SemaphoreType.DMA,
            pltpu.SemaphoreType.DMA,
            pltpu.SemaphoreType.DMA,
            pltpu.SemaphoreType.DMA,
            pltpu.VMEM_SHARED((N_ACC, AW), jnp.float32),
        ],
    )


def kernel(x, edge_index, W1, as1, ad1, bg1, Wr1, br1, gam1, bet1,
           W2, as2, ad2, bg2, Wr2, br2, gam2, bet2,
           W3, as3, ad3, bg3, Wr3, br3):
    loop = jnp.arange(N, dtype=jnp.int32)
    npad = E_PAD - E_TOT
    src = jnp.concatenate([edge_index[0], loop, jnp.zeros((npad,), jnp.int32)])
    dstg = jnp.concatenate([edge_index[1], loop, jnp.zeros((npad,), jnp.int32)])
    dsts = jnp.concatenate([edge_index[1], loop,
                            jnp.full((npad,), N, jnp.int32)])
    eidx = (jnp.stack([src, dstg, dsts])
            .reshape(3, KBLK, NS, EB).transpose(2, 1, 0, 3))

    bn_scale1 = (gam1 / jnp.sqrt(jnp.float32(1.0 + 1e-5))).reshape(1, D)
    bn_scale2 = (gam2 / jnp.sqrt(jnp.float32(1.0 + 1e-5))).reshape(1, D)
    bv1 = (bg1 + br1).reshape(1, D)
    bv2 = (bg2 + br2).reshape(1, D)
    bv3 = (bg3 + br3).reshape(1, D)
    p_sel, r8_sel = _sel_mats(NHEAD)
    _, r1_sel = _sel_mats(1)

    edge = _sc_edge()

    def _flat(t4):
        xls, sas, das, res = t4
        return (jnp.reshape(xls, (NC * N, HD)), jnp.reshape(sas, (NC * N, LB)),
                jnp.reshape(das, (NC * N, LB)), res)

    xl1, sa1, da1, res1 = _flat(_tc_first(x, W1, Wr1,
                                          _att_mats(as1), _att_mats(ad1)))
    np1 = edge(eidx, sa1, da1, xl1)
    xl2, sa2, da2, res2 = _flat(_tc_comb(np1, res1, bv1, bn_scale1,
                                         bet1.reshape(1, D), p_sel, r8_sel,
                                         W2, Wr2, _att_mats(as2),
                                         _att_mats(ad2)))
    np2 = edge(eidx, sa2, da2, xl2)
    xl3, sa3, da3, res3 = _flat(_tc_comb(np2, res2, bv2, bn_scale2,
                                         bet2.reshape(1, D), p_sel, r8_sel,
                                         W3, Wr3, _att_mats(as3),
                                         _att_mats(ad3)))
    np3 = edge(eidx, sa3, da3, xl3)
    return _tc_final(np3, res3, bv3, p_sel, r1_sel)


# speedup vs baseline: 2.0190x; 2.0190x over previous
import jax
import jax.numpy as jnp
from jax import lax
from jax.experimental import pallas as pl
from jax.experimental.pallas import tpu as pltpu
import jax.experimental.pallas.tpu_sc as plsc

N = 10000
E = 320000
D = 128
NHEAD = 8
HC = 16

NC = 2
NS = 16
NW = NC * NS
LB = 16

EB = 64
E_TOT = E + N
NBUF = 3
CH = 6
KBLK = 162
NCHUNK = KBLK // CH
E_PAD = KBLK * NW * EB
N_ACC = N + 8

ROWS_PER_TILE = N // NS
FLUSH_ROWS = (N // NS) // 8 * 8

NB = 1000
GRID = N // NB


def _att_mat(a):
    h, hc = a.shape
    if h == 1:
        return jnp.pad(jnp.tile(a.reshape(D, 1), (1, NHEAD)),
                       ((0, 0), (0, LB - NHEAD)))
    A = jnp.zeros((D, LB), jnp.float32)
    rows = jnp.arange(D)
    cols = jnp.repeat(jnp.arange(h), hc)
    return A.at[rows, cols].set(a.reshape(-1))


def _den_bcast_mat(hc):
    return (jnp.arange(LB)[:, None] == (jnp.arange(D) // hc)[None, :]).astype(
        jnp.float32)


def _tc_first_body(x_ref, w_ref, wr_ref, as_ref, ad_ref,
                   xl_ref, sa_ref, da_ref, res_ref):
    xv = x_ref[...]
    xl = jnp.dot(xv, w_ref[...].T, preferred_element_type=jnp.float32)
    xl_ref[...] = xl
    sa_ref[...] = jnp.dot(xl, as_ref[...], preferred_element_type=jnp.float32)
    da_ref[...] = jnp.dot(xl, ad_ref[...], preferred_element_type=jnp.float32)
    res_ref[...] = jnp.dot(xv, wr_ref[...].T, preferred_element_type=jnp.float32)


def _tc_comb_body(np_ref, dp_ref, res_ref, bvec_ref, scale_ref, shift_ref,
                  r_ref, w_ref, wr_ref, as_ref, ad_ref,
                  xl_ref, sa_ref, da_ref, res2_ref):
    num = np_ref[0] + np_ref[1]
    den = dp_ref[0] + dp_ref[1]
    denb = jnp.dot(den, r_ref[...], preferred_element_type=jnp.float32) + 1e-16
    hv = num / denb + res_ref[...] + bvec_ref[...]
    hv = jnp.maximum(hv * scale_ref[...] + shift_ref[...], 0.0)
    xl = jnp.dot(hv, w_ref[...].T, preferred_element_type=jnp.float32)
    xl_ref[...] = xl
    sa_ref[...] = jnp.dot(xl, as_ref[...], preferred_element_type=jnp.float32)
    da_ref[...] = jnp.dot(xl, ad_ref[...], preferred_element_type=jnp.float32)
    res2_ref[...] = jnp.dot(hv, wr_ref[...].T, preferred_element_type=jnp.float32)


def _tc_final_body(np_ref, dp_ref, res_ref, bvec_ref, r_ref, out_ref):
    num = np_ref[0] + np_ref[1]
    den = dp_ref[0] + dp_ref[1]
    denb = jnp.dot(den, r_ref[...], preferred_element_type=jnp.float32) + 1e-16
    out_ref[...] = num / denb + res_ref[...] + bvec_ref[...]


_FULL = lambda shape: pl.BlockSpec(shape, lambda i: tuple(0 for _ in shape))
_ROWS = pl.BlockSpec((NB, D), lambda i: (i, 0))
_ROWS16 = pl.BlockSpec((NB, LB), lambda i: (i, 0))
_PARTS = pl.BlockSpec((NC, NB, D), lambda i: (0, i, 0))
_PARTS16 = pl.BlockSpec((NC, NB, LB), lambda i: (0, i, 0))


def _tc_first(x, w, wr, a_s, a_d):
    return pl.pallas_call(
        _tc_first_body,
        grid=(GRID,),
        in_specs=[_ROWS, _FULL((D, D)), _FULL((D, D)),
                  _FULL((D, LB)), _FULL((D, LB))],
        out_specs=[_ROWS, _ROWS16, _ROWS16, _ROWS],
        out_shape=[jax.ShapeDtypeStruct((N, D), jnp.float32),
                   jax.ShapeDtypeStruct((N, LB), jnp.float32),
                   jax.ShapeDtypeStruct((N, LB), jnp.float32),
                   jax.ShapeDtypeStruct((N, D), jnp.float32)],
    )(x, w, wr, a_s, a_d)


def _tc_comb(npart, dpart, res, bvec, scale, shift, r, w, wr, a_s, a_d):
    return pl.pallas_call(
        _tc_comb_body,
        grid=(GRID,),
        in_specs=[_PARTS, _PARTS16, _ROWS, _FULL((1, D)), _FULL((1, D)),
                  _FULL((1, D)), _FULL((LB, D)), _FULL((D, D)), _FULL((D, D)),
                  _FULL((D, LB)), _FULL((D, LB))],
        out_specs=[_ROWS, _ROWS16, _ROWS16, _ROWS],
        out_shape=[jax.ShapeDtypeStruct((N, D), jnp.float32),
                   jax.ShapeDtypeStruct((N, LB), jnp.float32),
                   jax.ShapeDtypeStruct((N, LB), jnp.float32),
                   jax.ShapeDtypeStruct((N, D), jnp.float32)],
    )(npart, dpart, res, bvec, scale, shift, r, w, wr, a_s, a_d)


def _tc_final(npart, dpart, res, bvec, r):
    return pl.pallas_call(
        _tc_final_body,
        grid=(GRID,),
        in_specs=[_PARTS, _PARTS16, _ROWS, _FULL((1, D)), _FULL((LB, D))],
        out_specs=_ROWS,
        out_shape=jax.ShapeDtypeStruct((N, D), jnp.float32),
    )(npart, dpart, res, bvec, r)


def _sc_edge_body(eidx_hbm, sa_hbm, da_hbm, xl_hbm,
                  np_out, dp_out,
                  idx_v, sa_v, da_v, xr_v, wp_v,
                  semg0, semg1, semg2, sems0, sems1, sems2, semi,
                  num_acc, den_acc):
    cid = lax.axis_index("c")
    sid = lax.axis_index("s")
    wid = sid * NC + cid
    semg = (semg0, semg1, semg2)
    sems = (sems0, sems1, sems2)

    z16 = lax.iota(jnp.int32, LB).astype(jnp.float32) * 0.0

    for b in range(NBUF):
        def _zx(i, _, b=b):
            xr_v[b, i // 8, pl.ds((i % 8) * LB, LB)] = z16
            return 0
        lax.fori_loop(0, EB * 8, _zx, 0)

        def _zw(i, _, b=b):
            wp_v[b, i, :] = z16
            return 0
        lax.fori_loop(0, EB, _zw, 0)

    base_row = sid * ROWS_PER_TILE
    for r in range(9):
        pltpu.sync_copy(xr_v.at[0], num_acc.at[pl.ds(base_row + EB * r, EB)])
        pltpu.sync_copy(wp_v.at[0], den_acc.at[pl.ds(base_row + EB * r, EB)])
    rem = ROWS_PER_TILE - 9 * EB
    pltpu.sync_copy(xr_v.at[0, pl.ds(0, rem)],
                    num_acc.at[pl.ds(base_row + 9 * EB, rem)])
    pltpu.sync_copy(wp_v.at[0, pl.ds(0, rem)],
                    den_acc.at[pl.ds(base_row + 9 * EB, rem)])
    plsc.subcore_barrier()

    def _fetch_idx(c, p):
        pltpu.async_copy(eidx_hbm.at[wid, pl.ds(c * CH, CH)], idx_v.at[p],
                         semi)

    def _drain_idx(p):
        pltpu.make_async_copy(eidx_hbm.at[wid, pl.ds(0, CH)], idx_v.at[p],
                              semi).wait()

    def _issue_scat(b, p, j):
        pltpu.async_copy(xr_v.at[b], num_acc.at[idx_v.at[p, j, 1]],
                         sems[b], add=True)
        pltpu.async_copy(wp_v.at[b], den_acc.at[idx_v.at[p, j, 1]],
                         sems[b], add=True)

    def _drain_scat(b):
        pltpu.make_async_copy(xr_v.at[b], num_acc.at[idx_v.at[0, 0, 1]],
                              sems[b]).wait()
        pltpu.make_async_copy(wp_v.at[b], den_acc.at[idx_v.at[0, 0, 1]],
                              sems[b]).wait()

    def _issue_gath(b, p, j):
        pltpu.async_copy(sa_hbm.at[idx_v.at[p, j, 0]], sa_v.at[b], semg[b])
        pltpu.async_copy(da_hbm.at[idx_v.at[p, j, 1]], da_v.at[b], semg[b])
        pltpu.async_copy(xl_hbm.at[idx_v.at[p, j, 0]], xr_v.at[b], semg[b])

    def _drain_gath(b):
        pltpu.make_async_copy(sa_hbm.at[idx_v.at[0, 0, 0]], sa_v.at[b],
                              semg[b]).wait()
        pltpu.make_async_copy(da_hbm.at[idx_v.at[0, 0, 1]], da_v.at[b],
                              semg[b]).wait()
        pltpu.make_async_copy(xl_hbm.at[idx_v.at[0, 0, 0]], xr_v.at[b],
                              semg[b]).wait()

    def _compute(b):
        def _wpass(e, _):
            v = sa_v[b, e, :] + da_v[b, e, :]
            v = jnp.maximum(v, 0.2 * v)
            wp_v[b, e, :] = jnp.exp(v)
            return 0
        lax.fori_loop(0, EB, _wpass, 0, unroll=4)

        def _spass(e, _):
            v = wp_v[b, e, :]
            for h in range(NHEAD):
                s = v[h]
                xr_v[b, e, pl.ds(h * HC, HC)] = xr_v[b, e, pl.ds(h * HC, HC)] * s
            return 0
        lax.fori_loop(0, EB, _spass, 0, unroll=4)

    _fetch_idx(0, 0)
    _drain_idx(0)
    for b in range(NBUF):
        _issue_scat(b, 0, 0)
    _drain_scat(0)
    _issue_gath(0, 0, 0)
    _drain_scat(1)
    _issue_gath(1, 0, 1)

    def _chunk(c, _):
        p = lax.rem(c, 2)
        pn = lax.rem(c + 1, 2)
        cn = jnp.minimum(c + 1, NCHUNK - 1)
        for j in range(CH):
            b = j % NBUF
            _drain_gath(b)
            _compute(b)
            _issue_scat(b, p, j)
            if j == 1:
                _fetch_idx(cn, pn)
            if j == 3:
                _drain_idx(pn)
            bp = (b + 2) % NBUF
            _drain_scat(bp)
            if j < CH - 2:
                _issue_gath(bp, p, j + 2)
            else:
                _issue_gath(bp, pn, j + 2 - CH)
        return 0
    lax.fori_loop(0, NCHUNK, _chunk, 0)

    _drain_scat(NBUF - 1)
    _drain_gath(0)
    _drain_gath(1)
    plsc.subcore_barrier()

    rs = pl.ds(sid * FLUSH_ROWS, FLUSH_ROWS)
    pltpu.sync_copy(num_acc.at[rs], np_out.at[cid, rs])
    pltpu.sync_copy(den_acc.at[rs], dp_out.at[cid, rs])

    @pl.when(sid == NS - 1)
    def _tail():
        rs2 = pl.ds(NS * FLUSH_ROWS, N - NS * FLUSH_ROWS)
        pltpu.sync_copy(num_acc.at[rs2], np_out.at[cid, rs2])
        pltpu.sync_copy(den_acc.at[rs2], dp_out.at[cid, rs2])


def _sc_edge():
    return pl.kernel(
        _sc_edge_body,
        out_type=[jax.ShapeDtypeStruct((NC, N, D), jnp.float32),
                  jax.ShapeDtypeStruct((NC, N, LB), jnp.float32)],
        mesh=plsc.VectorSubcoreMesh(core_axis_name="c", subcore_axis_name="s",
                                    num_cores=NC, num_subcores=NS),
        compiler_params=pltpu.CompilerParams(use_tc_tiling_on_sc=False),
        scratch_types=[
            pltpu.VMEM((2, CH, 2, EB), jnp.int32),
            pltpu.VMEM((NBUF, EB, LB), jnp.float32),
            pltpu.VMEM((NBUF, EB, LB), jnp.float32),
            pltpu.VMEM((NBUF, EB, D), jnp.float32),
            pltpu.VMEM((NBUF, EB, LB), jnp.float32),
            pltpu.SemaphoreType.DMA,
            pltpu.SemaphoreType.DMA,
            pltpu.SemaphoreType.DMA,
            pltpu.SemaphoreType.DMA,
            pltpu.SemaphoreType.DMA,
            pltpu.SemaphoreType.DMA,
            pltpu.SemaphoreType.DMA,
            pltpu.VMEM_SHARED((N_ACC, D), jnp.float32),
            pltpu.VMEM_SHARED((N_ACC, LB), jnp.float32),
        ],
    )


def kernel(x, edge_index, W1, as1, ad1, bg1, Wr1, br1, gam1, bet1,
           W2, as2, ad2, bg2, Wr2, br2, gam2, bet2,
           W3, as3, ad3, bg3, Wr3, br3):
    loop = jnp.arange(N, dtype=jnp.int32)
    npad = E_PAD - E_TOT
    src = jnp.concatenate([edge_index[0], loop, jnp.zeros((npad,), jnp.int32)])
    dst = jnp.concatenate([edge_index[1], loop,
                           jnp.full((npad,), N, jnp.int32)])
    eidx = (jnp.stack([src, dst])
            .reshape(2, KBLK, NW, EB).transpose(2, 1, 0, 3))

    bn_scale1 = (gam1 / jnp.sqrt(jnp.float32(1.0 + 1e-5))).reshape(1, D)
    bn_scale2 = (gam2 / jnp.sqrt(jnp.float32(1.0 + 1e-5))).reshape(1, D)
    bv1 = (bg1 + br1).reshape(1, D)
    bv2 = (bg2 + br2).reshape(1, D)
    bv3 = (bg3 + br3).reshape(1, D)
    r8 = _den_bcast_mat(HC)
    r1 = _den_bcast_mat(D)

    edge = _sc_edge()

    xl1, sa1, da1, res1 = _tc_first(x, W1, Wr1, _att_mat(as1), _att_mat(ad1))
    np1, dp1 = edge(eidx, sa1, da1, xl1)
    xl2, sa2, da2, res2 = _tc_comb(np1, dp1, res1, bv1, bn_scale1,
                                   bet1.reshape(1, D), r8, W2, Wr2,
                                   _att_mat(as2), _att_mat(ad2))
    np2, dp2 = edge(eidx, sa2, da2, xl2)
    xl3, sa3, da3, res3 = _tc_comb(np2, dp2, res2, bv2, bn_scale2,
                                   bet2.reshape(1, D), r8, W3, Wr3,
                                   _att_mat(as3), _att_mat(ad3))
    np3, dp3 = edge(eidx, sa3, da3, xl3)
    return _tc_final(np3, dp3, res3, bv3, r1)
